# Initial kernel scaffold; baseline (speedup 1.0000x reference)
import jax
import jax.numpy as jnp
from jax.experimental import pallas as pl


def kernel(x, head_W, head_b):
    raise NotImplementedError("write your pallas kernel here")


# speedup vs baseline: 38.4370x; 38.4370x over previous
import functools

import jax
import jax.numpy as jnp
from jax.experimental import pallas as pl

_K = 3
_ITERS = 50
_BAGS_LEN = 1042
_CLUS_LEN = 961
_TGT_LEN = _BAGS_LEN - _CLUS_LEN
_D = 768
_B = 4

_HIGHEST = jax.lax.Precision.HIGHEST


def _tic_mil_kernel(clus_ref, tgt_ref, u_ref, w_ref, b_ref,
                    out_ref, mind_ref, nonmind_ref):
    f32 = jnp.float32

    cl = [clus_ref[b] for b in range(_B)]

    centers0 = []
    for b in range(_B):
        col_max = jnp.max(cl[b], axis=0)
        col_min = jnp.min(cl[b], axis=0)
        centers0.append(col_max[None, :] + u_ref[b] * (col_min[None, :] - col_max[None, :]))

    def assign_from_centers(b, centers_b):
        dists = []
        for k in range(_K):
            diff = cl[b] - centers_b[k][None, :]
            dists.append(jnp.sqrt(jnp.sum(diff * diff, axis=1)))
        best = dists[0]
        idx = jnp.zeros((_CLUS_LEN,), dtype=jnp.int32)
        for k in range(1, _K):
            m = dists[k] < best
            idx = jnp.where(m, k, idx)
            best = jnp.where(m, dists[k], best)
        return idx

    def body(state):
        it, _stable, centers, prev = state
        new_assign = []
        new_centers = []
        for b in range(_B):
            a = assign_from_centers(b, centers[b])
            new_assign.append(a)
            oh = (jax.lax.broadcasted_iota(jnp.int32, (_K, _CLUS_LEN), 0)
                  == a[None, :]).astype(f32)
            sums = jax.lax.dot_general(oh, cl[b], (((1,), (0,)), ((), ())),
                                       precision=_HIGHEST,
                                       preferred_element_type=f32)
            counts = jnp.sum(oh, axis=1)
            newc = jnp.where(counts[:, None] > 0,
                             sums / jnp.maximum(counts, 1.0)[:, None],
                             centers[b])
            new_centers.append(newc)
        stable = jnp.bool_(True)
        for b in range(_B):
            stable = stable & jnp.all(new_assign[b] == prev[b])
        return (it + 1, stable, tuple(new_centers), tuple(new_assign))

    def cond(state):
        it, stable, _c, _a = state
        return (it < _ITERS) & jnp.logical_not(stable)

    init_assign = tuple(jnp.full((_CLUS_LEN,), -1, dtype=jnp.int32)
                        for _ in range(_B))
    _it, _st, _centers, assign = jax.lax.while_loop(
        cond, body, (jnp.int32(0), jnp.bool_(False), tuple(centers0), init_assign))

    inv_cd = f32(1.0) / f32(_CLUS_LEN * _D)
    pooled = []
    dmins = []
    dsums = []
    for b in range(_B):
        tg = tgt_ref[b]
        t_mean = jnp.sum(tg) / f32(_TGT_LEN * _D)
        rs = jnp.sum(cl[b], axis=1)
        dis = []
        for k in range(_K):
            mask = (assign[b] == k).astype(f32)
            cnt = jnp.sum(mask)
            csum = jnp.sum(mask * rs)
            cmean = jnp.where(cnt > 0,
                              csum / jnp.maximum(cnt * f32(_D), 1.0),
                              f32(0.0))
            dis.append(jnp.abs(t_mean - cmean))
        dmins.append(jnp.minimum(jnp.minimum(dis[0], dis[1]), dis[2]))
        dsums.append(dis[0] + dis[1] + dis[2])
        scale = jnp.zeros((_CLUS_LEN,), dtype=f32)
        for k in range(_K):
            scale = scale + (assign[b] == k).astype(f32) * (f32(1.0) - dis[k])
        clus_scaled = cl[b] * scale[:, None]
        pooled.append(((jnp.sum(clus_scaled, axis=0) + jnp.sum(tg, axis=0))
                       / f32(_BAGS_LEN))[None, :])

    feat = jnp.concatenate(pooled, axis=0)
    out = jax.lax.dot_general(feat, w_ref[...], (((1,), (1,)), ((), ())),
                              precision=_HIGHEST,
                              preferred_element_type=f32)
    out_ref[...] = out + b_ref[...]

    min_dis = (((dmins[0] + dmins[1]) + dmins[2]) + dmins[3]) / f32(_B)
    s_all = (((dsums[0] + dsums[1]) + dsums[2]) + dsums[3])
    m_all = (((dmins[0] + dmins[1]) + dmins[2]) + dmins[3])
    non_min_dis = (s_all - m_all) / f32(_B)
    mind_ref[...] = jnp.reshape(min_dis, (1, 1))
    nonmind_ref[...] = jnp.reshape(non_min_dis, (1, 1))


@functools.partial(jax.jit, static_argnames=("interpret",))
def kernel(x, head_W, head_b, interpret=False):
    B = x.shape[0] // _BAGS_LEN
    y = jnp.reshape(x, (B, _BAGS_LEN, _D))
    clus = y[:, :_CLUS_LEN, :]
    tgt = y[:, _CLUS_LEN:, :]
    u = jnp.stack([
        jax.random.uniform(jax.random.fold_in(jax.random.key(42), i),
                           (_K, _D), dtype=jnp.float32)
        for i in range(B)])
    out, mind, nonmind = pl.pallas_call(
        _tic_mil_kernel,
        out_shape=(
            jax.ShapeDtypeStruct((B, _K), jnp.float32),
            jax.ShapeDtypeStruct((1, 1), jnp.float32),
            jax.ShapeDtypeStruct((1, 1), jnp.float32),
        ),
        interpret=interpret,
    )(clus, tgt, u, head_W, jnp.reshape(head_b, (1, _K)))
    return (out, jnp.reshape(mind, (1,)), jnp.reshape(nonmind, (1,)))


# speedup vs baseline: 46.1839x; 1.2015x over previous
import functools

import jax
import jax.numpy as jnp
from jax.experimental import pallas as pl
from jax.experimental.pallas import tpu as pltpu

_K = 3
_ITERS = 50
_BAGS_LEN = 1042
_CLUS_LEN = 961
_TGT_LEN = _BAGS_LEN - _CLUS_LEN
_D = 768
_B = 4
_BH = 2

_HIGHEST = jax.lax.Precision.HIGHEST


def _tic_mil_kernel(clus_ref, tgt_ref, u_ref, w_ref, b_ref,
                    out_ref, mind_ref, nonmind_ref):
    f32 = jnp.float32
    bf16 = jnp.bfloat16

    cl = [clus_ref[b] for b in range(_BH)]

    cl_hi, cl_mid, cl_lo = [], [], []
    for b in range(_BH):
        hi = cl[b].astype(bf16)
        r1 = cl[b] - hi.astype(f32)
        mid = r1.astype(bf16)
        lo = (r1 - mid.astype(f32)).astype(bf16)
        cl_hi.append(hi)
        cl_mid.append(mid)
        cl_lo.append(lo)

    centers0 = []
    for b in range(_BH):
        col_max = jnp.max(cl[b], axis=0)
        col_min = jnp.min(cl[b], axis=0)
        centers0.append(col_max[None, :] + u_ref[b] * (col_min[None, :] - col_max[None, :]))

    def assign_from_centers(b, centers_b):
        dists = []
        for k in range(_K):
            diff = cl[b] - centers_b[k][None, :]
            dists.append(jnp.sqrt(jnp.sum(diff * diff, axis=1)))
        best = dists[0]
        idx = jnp.zeros((_CLUS_LEN,), dtype=jnp.int32)
        for k in range(1, _K):
            m = dists[k] < best
            idx = jnp.where(m, k, idx)
            best = jnp.where(m, dists[k], best)
        return idx

    def _seg_matmul(oh_bf, b):
        dn = (((1,), (0,)), ((), ()))
        s = jax.lax.dot_general(oh_bf, cl_hi[b], dn, preferred_element_type=f32)
        s = s + jax.lax.dot_general(oh_bf, cl_mid[b], dn, preferred_element_type=f32)
        s = s + jax.lax.dot_general(oh_bf, cl_lo[b], dn, preferred_element_type=f32)
        return s

    def body(state):
        it, _stable, centers, prev = state
        new_assign = []
        new_centers = []
        for b in range(_BH):
            a = assign_from_centers(b, centers[b])
            new_assign.append(a)
            ohm = (jax.lax.broadcasted_iota(jnp.int32, (_K, _CLUS_LEN), 0)
                   == a[None, :])
            sums = _seg_matmul(ohm.astype(bf16), b)
            counts = jnp.sum(ohm.astype(f32), axis=1)
            newc = jnp.where(counts[:, None] > 0,
                             sums / jnp.maximum(counts, 1.0)[:, None],
                             centers[b])
            new_centers.append(newc)
        stable = jnp.bool_(True)
        for b in range(_BH):
            stable = stable & jnp.all(new_assign[b] == prev[b])
        return (it + 1, stable, tuple(new_centers), tuple(new_assign))

    def cond(state):
        it, stable, _c, _a = state
        return (it < _ITERS) & jnp.logical_not(stable)

    init_assign = tuple(jnp.full((_CLUS_LEN,), -1, dtype=jnp.int32)
                        for _ in range(_BH))
    _it, _st, _centers, assign = jax.lax.while_loop(
        cond, body, (jnp.int32(0), jnp.bool_(False), tuple(centers0), init_assign))

    pooled = []
    dmins = []
    dsums = []
    for b in range(_BH):
        tg = tgt_ref[b]
        t_mean = jnp.sum(tg) / f32(_TGT_LEN * _D)
        rs = jnp.sum(cl[b], axis=1)
        dis = []
        for k in range(_K):
            mask = (assign[b] == k).astype(f32)
            cnt = jnp.sum(mask)
            csum = jnp.sum(mask * rs)
            cmean = jnp.where(cnt > 0,
                              csum / jnp.maximum(cnt * f32(_D), 1.0),
                              f32(0.0))
            dis.append(jnp.abs(t_mean - cmean))
        dmins.append(jnp.minimum(jnp.minimum(dis[0], dis[1]), dis[2]))
        dsums.append(dis[0] + dis[1] + dis[2])
        scale = jnp.zeros((_CLUS_LEN,), dtype=f32)
        for k in range(_K):
            scale = scale + (assign[b] == k).astype(f32) * (f32(1.0) - dis[k])
        clus_scaled = cl[b] * scale[:, None]
        pooled.append(((jnp.sum(clus_scaled, axis=0) + jnp.sum(tg, axis=0))
                       / f32(_BAGS_LEN))[None, :])

    feat = jnp.concatenate(pooled, axis=0)
    out = jax.lax.dot_general(feat, w_ref[...], (((1,), (1,)), ((), ())),
                              precision=_HIGHEST,
                              preferred_element_type=f32)
    out_ref[...] = jnp.reshape(out + b_ref[...], (1, _BH, _K))

    mind_ref[...] = jnp.reshape(dmins[0] + dmins[1], (1, 1, 1))
    nonmind_ref[...] = jnp.reshape((dsums[0] + dsums[1])
                                   - (dmins[0] + dmins[1]), (1, 1, 1))


@functools.partial(jax.jit, static_argnames=("interpret",))
def kernel(x, head_W, head_b, interpret=False):
    B = x.shape[0] // _BAGS_LEN
    y = jnp.reshape(x, (B, _BAGS_LEN, _D))
    clus = y[:, :_CLUS_LEN, :]
    tgt = y[:, _CLUS_LEN:, :]
    u = jnp.stack([
        jax.random.uniform(jax.random.fold_in(jax.random.key(42), i),
                           (_K, _D), dtype=jnp.float32)
        for i in range(B)])
    n_prog = B // _BH
    out, mind, nonmind = pl.pallas_call(
        _tic_mil_kernel,
        grid=(n_prog,),
        in_specs=[
            pl.BlockSpec((_BH, _CLUS_LEN, _D), lambda i: (i, 0, 0)),
            pl.BlockSpec((_BH, _TGT_LEN, _D), lambda i: (i, 0, 0)),
            pl.BlockSpec((_BH, _K, _D), lambda i: (i, 0, 0)),
            pl.BlockSpec((_K, _D), lambda i: (0, 0)),
            pl.BlockSpec((1, _K), lambda i: (0, 0)),
        ],
        out_specs=(
            pl.BlockSpec((1, _BH, _K), lambda i: (i, 0, 0)),
            pl.BlockSpec((1, 1, 1), lambda i: (i, 0, 0)),
            pl.BlockSpec((1, 1, 1), lambda i: (i, 0, 0)),
        ),
        out_shape=(
            jax.ShapeDtypeStruct((n_prog, _BH, _K), jnp.float32),
            jax.ShapeDtypeStruct((n_prog, 1, 1), jnp.float32),
            jax.ShapeDtypeStruct((n_prog, 1, 1), jnp.float32),
        ),
        compiler_params=pltpu.CompilerParams(
            dimension_semantics=("parallel",)),
        interpret=interpret,
    )(clus, tgt, u, head_W, jnp.reshape(head_b, (1, _K)))
    out = jnp.reshape(out, (B, _K))
    min_dis = jnp.reshape((mind[0, 0, 0] + mind[1, 0, 0]) / jnp.float32(B), (1,))
    non_min_dis = jnp.reshape((nonmind[0, 0, 0] + nonmind[1, 0, 0]) / jnp.float32(B), (1,))
    return (out, min_dis, non_min_dis)
